# Initial kernel scaffold; baseline (speedup 1.0000x reference)
import jax
import jax.numpy as jnp
from jax.experimental import pallas as pl


def kernel(x, edge_index, att_l0, att_r0, W0, b0, att_l1, att_r1, W1, b1, att_l2, att_r2, W2, b2, cls_W, cls_b):
    raise NotImplementedError("write your pallas kernel here")


# speedup vs baseline: 17.4133x; 17.4133x over previous
import functools

import jax
import jax.numpy as jnp
from jax import lax
from jax.experimental import pallas as pl
from jax.experimental.pallas import tpu as pltpu
from jax.experimental.pallas import tpu_sc as plsc

N = 10000
E = 320000
D = 128
N_CLS = 16
EPS = 0.1

NC = 2
NS = 16
NW = NC * NS
EW = E // NW
K = 80
NCH = EW // K
RPT = N // NS
ZR = 125

_sc_mesh = plsc.VectorSubcoreMesh(core_axis_name="c", subcore_axis_name="s")


@functools.partial(
    pl.kernel,
    out_type=jax.ShapeDtypeStruct((NC, NS, RPT, 16), jnp.float32),
    mesh=_sc_mesh,
    scratch_types=[
        pltpu.VMEM((K,), jnp.int32),
        pltpu.VMEM((K, 16), jnp.float32),
        pltpu.VMEM((ZR, 16), jnp.float32),
        pltpu.VMEM_SHARED((N, 16), jnp.float32),
    ],
    compiler_params=pltpu.CompilerParams(needs_layout_passes=False),
)
def _deg_kernel(dst_hbm, out_hbm, idx_v, ones_v, zer_v, acc):
    c = lax.axis_index("c")
    s = lax.axis_index("s")
    wid = s * NC + c
    ones16 = jnp.full((16,), 1.0, jnp.float32)
    zeros16 = jnp.zeros((16,), jnp.float32)

    def fill(i, _):
        ones_v[i, :] = ones16
        zer_v[i, :] = zeros16
        return 0

    lax.fori_loop(0, ZR, fill, 0)
    for t in range(RPT // ZR):
        pltpu.sync_copy(zer_v, acc.at[pl.ds(s * RPT + t * ZR, ZR)])
    plsc.subcore_barrier()

    base = wid * EW

    def body(i, _):
        pltpu.sync_copy(dst_hbm.at[pl.ds(base + i * K, K)], idx_v)
        pltpu.sync_copy(ones_v, acc.at[idx_v], add=True)
        return 0

    lax.fori_loop(0, NCH, body, 0)
    plsc.subcore_barrier()
    pltpu.sync_copy(acc.at[pl.ds(s * RPT, RPT)], out_hbm.at[c, s])


@functools.partial(
    pl.kernel,
    out_type=jax.ShapeDtypeStruct((NC, NS, RPT, D), jnp.float32),
    mesh=_sc_mesh,
    scratch_types=[
        pltpu.VMEM((N,), jnp.float32),
        pltpu.VMEM((N,), jnp.float32),
        pltpu.VMEM((N,), jnp.float32),
        pltpu.VMEM((K,), jnp.int32),
        pltpu.VMEM((K,), jnp.int32),
        pltpu.VMEM((K,), jnp.float32),
        pltpu.VMEM((K, D), jnp.float32),
        pltpu.VMEM_SHARED((N, D), jnp.float32),
    ],
    compiler_params=pltpu.CompilerParams(needs_layout_passes=False),
)
def _agg_kernel(h_hbm, al_hbm, ar_hbm, dis_hbm, src_hbm, dst_hbm, out_hbm,
                al_v, ar_v, dis_v, src_v, dst_v, c_v, rows_v, acc):
    c = lax.axis_index("c")
    s = lax.axis_index("s")
    wid = s * NC + c
    pltpu.sync_copy(al_hbm, al_v)
    pltpu.sync_copy(ar_hbm, ar_v)
    pltpu.sync_copy(dis_hbm, dis_v)

    zeros16 = jnp.zeros((16,), jnp.float32)

    def fillz(i, _):
        for l in range(D // 16):
            rows_v[i, pl.ds(l * 16, 16)] = zeros16
        return 0

    lax.fori_loop(0, K, fillz, 0)
    for t in range(RPT // K):
        pltpu.sync_copy(rows_v, acc.at[pl.ds(s * RPT + t * K, K)])
    pltpu.sync_copy(rows_v.at[pl.ds(0, RPT % K)],
                    acc.at[pl.ds(s * RPT + (RPT // K) * K, RPT % K)])
    plsc.subcore_barrier()

    base = wid * EW

    def chunk(i, _):
        pltpu.sync_copy(src_hbm.at[pl.ds(base + i * K, K)], src_v)
        pltpu.sync_copy(dst_hbm.at[pl.ds(base + i * K, K)], dst_v)
        pltpu.sync_copy(h_hbm.at[src_v], rows_v)
        for g in range(K // 16):
            s16 = src_v[pl.ds(g * 16, 16)]
            d16 = dst_v[pl.ds(g * 16, 16)]
            a = plsc.load_gather(al_v, [d16]) + plsc.load_gather(ar_v, [s16])
            w = plsc.load_gather(dis_v, [s16]) * plsc.load_gather(dis_v, [d16])
            ex = jnp.exp(a * 2.0)
            tanh_a = 1.0 - 2.0 / (ex + 1.0)
            c_v[pl.ds(g * 16, 16)] = tanh_a * w

        def scale(g, _):
            c16 = c_v[pl.ds(g * 16, 16)]
            for j in range(16):
                cj = c16[j]
                row = g * 16 + j
                for l in range(D // 16):
                    rows_v[row, pl.ds(l * 16, 16)] = (
                        rows_v[row, pl.ds(l * 16, 16)] * cj)
            return 0

        lax.fori_loop(0, K // 16, scale, 0)
        pltpu.sync_copy(rows_v, acc.at[dst_v], add=True)
        return 0

    lax.fori_loop(0, NCH, chunk, 0)
    plsc.subcore_barrier()
    pltpu.sync_copy(acc.at[pl.ds(s * RPT, RPT)], out_hbm.at[c, s])


BM = 2000


def _prep_body(degacc_ref, x_ref, attl_ref, attr_ref,
               dis_ref, al_ref, ar_ref, selfc_ref):
    cnt = jnp.sum(degacc_ref[...], axis=2, keepdims=True) * (1.0 / 16.0)
    deg = 1.0 + cnt[0] + cnt[1]
    dis = lax.rsqrt(deg)
    al = jnp.sum(x_ref[...] * attl_ref[...][None, :], axis=1, keepdims=True)
    ar = jnp.sum(x_ref[...] * attr_ref[...][None, :], axis=1, keepdims=True)
    dis_ref[...] = dis
    al_ref[...] = al
    ar_ref[...] = ar
    selfc_ref[...] = jnp.tanh(al + ar) * dis * dis


_prep = pl.pallas_call(
    _prep_body,
    grid=(N // BM,),
    in_specs=[
        pl.BlockSpec((2, BM, 16), lambda i: (0, i, 0)),
        pl.BlockSpec((BM, D), lambda i: (i, 0)),
        pl.BlockSpec((D,), lambda i: (0,)),
        pl.BlockSpec((D,), lambda i: (0,)),
    ],
    out_specs=(
        pl.BlockSpec((BM, 1), lambda i: (i, 0)),
        pl.BlockSpec((BM, 1), lambda i: (i, 0)),
        pl.BlockSpec((BM, 1), lambda i: (i, 0)),
        pl.BlockSpec((BM, 1), lambda i: (i, 0)),
    ),
    out_shape=(
        jax.ShapeDtypeStruct((N, 1), jnp.float32),
        jax.ShapeDtypeStruct((N, 1), jnp.float32),
        jax.ShapeDtypeStruct((N, 1), jnp.float32),
        jax.ShapeDtypeStruct((N, 1), jnp.float32),
    ),
)


def _mix_body(agg_ref, h_ref, selfc_ref, dis_ref, W_ref, b_ref,
              attl_ref, attr_ref, hn_ref, al_ref, ar_ref, selfcn_ref):
    o = agg_ref[0] + agg_ref[1] + h_ref[...] * (selfc_ref[...] + EPS)
    hn = jnp.dot(o, W_ref[...], preferred_element_type=jnp.float32)
    hn = hn + b_ref[...][None, :]
    hn_ref[...] = hn
    al = jnp.sum(hn * attl_ref[...][None, :], axis=1, keepdims=True)
    ar = jnp.sum(hn * attr_ref[...][None, :], axis=1, keepdims=True)
    al_ref[...] = al
    ar_ref[...] = ar
    dis = dis_ref[...]
    selfcn_ref[...] = jnp.tanh(al + ar) * dis * dis


_mix = pl.pallas_call(
    _mix_body,
    grid=(N // BM,),
    in_specs=[
        pl.BlockSpec((2, BM, D), lambda i: (0, i, 0)),
        pl.BlockSpec((BM, D), lambda i: (i, 0)),
        pl.BlockSpec((BM, 1), lambda i: (i, 0)),
        pl.BlockSpec((BM, 1), lambda i: (i, 0)),
        pl.BlockSpec((D, D), lambda i: (0, 0)),
        pl.BlockSpec((D,), lambda i: (0,)),
        pl.BlockSpec((D,), lambda i: (0,)),
        pl.BlockSpec((D,), lambda i: (0,)),
    ],
    out_specs=(
        pl.BlockSpec((BM, D), lambda i: (i, 0)),
        pl.BlockSpec((BM, 1), lambda i: (i, 0)),
        pl.BlockSpec((BM, 1), lambda i: (i, 0)),
        pl.BlockSpec((BM, 1), lambda i: (i, 0)),
    ),
    out_shape=(
        jax.ShapeDtypeStruct((N, D), jnp.float32),
        jax.ShapeDtypeStruct((N, 1), jnp.float32),
        jax.ShapeDtypeStruct((N, 1), jnp.float32),
        jax.ShapeDtypeStruct((N, 1), jnp.float32),
    ),
)


def _fin_body(agg_ref, h_ref, selfc_ref, W_ref, b_ref, cW_ref, cb_ref,
              out_ref):
    o = agg_ref[0] + agg_ref[1] + h_ref[...] * (selfc_ref[...] + EPS)
    hn = jnp.dot(o, W_ref[...], preferred_element_type=jnp.float32)
    hn = hn + b_ref[...][None, :]
    out_ref[...] = (jnp.dot(hn, cW_ref[...], preferred_element_type=jnp.float32)
                    + cb_ref[...][None, :])


_fin = pl.pallas_call(
    _fin_body,
    grid=(N // BM,),
    in_specs=[
        pl.BlockSpec((2, BM, D), lambda i: (0, i, 0)),
        pl.BlockSpec((BM, D), lambda i: (i, 0)),
        pl.BlockSpec((BM, 1), lambda i: (i, 0)),
        pl.BlockSpec((D, D), lambda i: (0, 0)),
        pl.BlockSpec((D,), lambda i: (0,)),
        pl.BlockSpec((D, N_CLS), lambda i: (0, 0)),
        pl.BlockSpec((N_CLS,), lambda i: (0,)),
    ],
    out_specs=pl.BlockSpec((BM, N_CLS), lambda i: (i, 0)),
    out_shape=jax.ShapeDtypeStruct((N, N_CLS), jnp.float32),
)


def kernel(x, edge_index, att_l0, att_r0, W0, b0, att_l1, att_r1, W1, b1,
           att_l2, att_r2, W2, b2, cls_W, cls_b):
    src = edge_index[0]
    dst = edge_index[1]
    degacc = _deg_kernel(dst).reshape(NC, N, 16)
    dis, al, ar, selfc = _prep(degacc, x, att_l0, att_r0)
    dis_f = dis.reshape(N)
    h = x
    layers = ((W0, b0, att_l1, att_r1), (W1, b1, att_l2, att_r2))
    for _, (W, b, attl_n, attr_n) in enumerate(layers):
        agg = _agg_kernel(h, al.reshape(N), ar.reshape(N), dis_f,
                          src, dst).reshape(NC, N, D)
        h, al, ar, selfc = _mix(agg, h, selfc, dis, W, b, attl_n, attr_n)
    agg = _agg_kernel(h, al.reshape(N), ar.reshape(N), dis_f,
                      src, dst).reshape(NC, N, D)
    return _fin(agg, h, selfc, W2, b2, cls_W, cls_b)
